# baseline (speedup 1.0000x reference)
import functools

import jax
import jax.numpy as jnp
from jax.experimental import pallas as pl
from jax.experimental.pallas import tpu as pltpu

NUM_EMB = 1024
DIM = 64
COMMIT = 0.25


def _vq_kernel(x_ref, e_ref, q_ref, idx_ref, loss_ref):
    c, h, w = x_ref.shape[1:]
    p = h * w
    x = x_ref[0].reshape(c, p)
    e = e_ref[...]

    x_sq = jnp.sum(x * x, axis=0, keepdims=True)
    e_sq = jnp.sum(e * e, axis=1, keepdims=True)
    s2 = jax.lax.dot_general(
        e + e, x, (((1,), (0,)), ((), ())),
        preferred_element_type=jnp.float32)
    dist = (x_sq + e_sq) - s2

    m = jnp.min(dist, axis=0, keepdims=True)
    iota_j = jax.lax.broadcasted_iota(jnp.int32, dist.shape, 0)
    idx = jnp.min(jnp.where(dist == m, iota_j, jnp.int32(NUM_EMB)),
                  axis=0, keepdims=True)

    onehot = (iota_j == idx).astype(jnp.float32)
    q = jax.lax.dot_general(
        e, onehot, (((0,), (0,)), ((), ())),
        preferred_element_type=jnp.float32)

    loss = jnp.sum(m) * ((1.0 + COMMIT) / (DIM * p))

    q_ref[0] = q.reshape(c, h, w)
    idx_ref[0] = idx
    loss_ref[0] = jnp.full((1, 128), loss, dtype=jnp.float32)


@functools.partial(jax.jit, static_argnames=())
def kernel(inputs, embedding):
    b, c, h, w = inputs.shape

    q, idx, loss = pl.pallas_call(
        _vq_kernel,
        grid=(b,),
        in_specs=[
            pl.BlockSpec((1, c, h, w), lambda i: (i, 0, 0, 0)),
            pl.BlockSpec((NUM_EMB, DIM), lambda i: (0, 0)),
        ],
        out_specs=[
            pl.BlockSpec((1, c, h, w), lambda i: (i, 0, 0, 0)),
            pl.BlockSpec((1, 1, h * w), lambda i: (i, 0, 0)),
            pl.BlockSpec((1, 1, 128), lambda i: (i, 0, 0)),
        ],
        out_shape=[
            jax.ShapeDtypeStruct((b, c, h, w), jnp.float32),
            jax.ShapeDtypeStruct((b, 1, h * w), jnp.int32),
            jax.ShapeDtypeStruct((b, 1, 128), jnp.float32),
        ],
        compiler_params=pltpu.CompilerParams(
            dimension_semantics=("parallel",)),
    )(inputs, embedding)

    return (q, idx.reshape(b, h, w), loss[:, 0, 0])


# speedup vs baseline: 1.1887x; 1.1887x over previous
import functools

import jax
import jax.numpy as jnp
from jax.experimental import pallas as pl
from jax.experimental.pallas import tpu as pltpu

NUM_EMB = 1024
DIM = 64
COMMIT = 0.25


def _vq_kernel(x_ref, e_ref, q_ref, idx_ref, loss_ref):
    x = x_ref[0]
    e = e_ref[...]
    p = x.shape[0]

    x_sq = jnp.sum(x * x, axis=1, keepdims=True)
    e_sq = jnp.sum(e * e, axis=1, keepdims=True)
    e_sq_row = e_sq.reshape(1, NUM_EMB)
    s2 = jax.lax.dot_general(
        x, e + e, (((1,), (1,)), ((), ())),
        preferred_element_type=jnp.float32)
    dist = (x_sq + e_sq_row) - s2

    m = jnp.min(dist, axis=1, keepdims=True)
    iota_j = jax.lax.broadcasted_iota(jnp.int32, dist.shape, 1)
    idx = jnp.min(jnp.where(dist == m, iota_j, jnp.int32(NUM_EMB)),
                  axis=1, keepdims=True)

    onehot = (iota_j == idx).astype(jnp.float32)
    q = jax.lax.dot_general(
        onehot, e, (((1,), (0,)), ((), ())),
        preferred_element_type=jnp.float32)

    loss = jnp.sum(m) * ((1.0 + COMMIT) / (DIM * p))

    q_ref[0] = q
    idx_ref[0] = idx.reshape(1, p)
    loss_ref[0] = jnp.full((1, 128), loss, dtype=jnp.float32)


@functools.partial(jax.jit, static_argnames=())
def kernel(inputs, embedding):
    b, c, h, w = inputs.shape
    p = h * w
    x = jnp.transpose(inputs, (0, 2, 3, 1)).reshape(b, p, c)

    q, idx, loss = pl.pallas_call(
        _vq_kernel,
        grid=(b,),
        in_specs=[
            pl.BlockSpec((1, p, c), lambda i: (i, 0, 0)),
            pl.BlockSpec((NUM_EMB, DIM), lambda i: (0, 0)),
        ],
        out_specs=[
            pl.BlockSpec((1, p, c), lambda i: (i, 0, 0)),
            pl.BlockSpec((1, 1, p), lambda i: (i, 0, 0)),
            pl.BlockSpec((1, 1, 128), lambda i: (i, 0, 0)),
        ],
        out_shape=[
            jax.ShapeDtypeStruct((b, p, c), jnp.float32),
            jax.ShapeDtypeStruct((b, 1, p), jnp.int32),
            jax.ShapeDtypeStruct((b, 1, 128), jnp.float32),
        ],
        compiler_params=pltpu.CompilerParams(
            dimension_semantics=("parallel",)),
    )(x, embedding)

    quantized_st = jnp.transpose(q.reshape(b, h, w, c), (0, 3, 1, 2))
    enc_idx = idx.reshape(b, h, w)
    loss_out = loss[:, 0, 0]
    return (quantized_st, enc_idx, loss_out)


# speedup vs baseline: 1.7490x; 1.4714x over previous
import functools

import jax
import jax.numpy as jnp
from jax.experimental import pallas as pl
from jax.experimental.pallas import tpu as pltpu

NUM_EMB = 1024
DIM = 64
COMMIT = 0.25


def _vq_kernel(x_ref, e_ref, q_ref, idx_ref, loss_ref):
    xp = x_ref[0]
    e = e_ref[...]
    p = xp.shape[0]

    x = jnp.transpose(xp, (1, 0))

    x_sq = jnp.sum(x * x, axis=0, keepdims=True)
    e_sq = jnp.sum(e * e, axis=1, keepdims=True)
    s2 = jax.lax.dot_general(
        e + e, x, (((1,), (0,)), ((), ())),
        preferred_element_type=jnp.float32)
    dist = (x_sq + e_sq) - s2

    m = jnp.min(dist, axis=0, keepdims=True)
    iota_j = jax.lax.broadcasted_iota(jnp.int32, dist.shape, 0)
    idx = jnp.min(jnp.where(dist == m, iota_j, jnp.int32(NUM_EMB)),
                  axis=0, keepdims=True)

    onehot = (iota_j == idx).astype(jnp.float32)
    q = jax.lax.dot_general(
        e, onehot, (((0,), (0,)), ((), ())),
        preferred_element_type=jnp.float32)

    loss = jnp.sum(m) * ((1.0 + COMMIT) / (DIM * p))

    q_ref[0] = jnp.transpose(q, (1, 0))
    idx_ref[0] = idx
    loss_ref[0] = jnp.full((1, 128), loss, dtype=jnp.float32)


@functools.partial(jax.jit, static_argnames=())
def kernel(inputs, embedding):
    b, c, h, w = inputs.shape
    p = h * w
    x = jnp.transpose(inputs, (0, 2, 3, 1)).reshape(b, p, c)

    q, idx, loss = pl.pallas_call(
        _vq_kernel,
        grid=(b,),
        in_specs=[
            pl.BlockSpec((1, p, c), lambda i: (i, 0, 0)),
            pl.BlockSpec((NUM_EMB, DIM), lambda i: (0, 0)),
        ],
        out_specs=[
            pl.BlockSpec((1, p, c), lambda i: (i, 0, 0)),
            pl.BlockSpec((1, 1, p), lambda i: (i, 0, 0)),
            pl.BlockSpec((1, 1, 128), lambda i: (i, 0, 0)),
        ],
        out_shape=[
            jax.ShapeDtypeStruct((b, p, c), jnp.float32),
            jax.ShapeDtypeStruct((b, 1, p), jnp.int32),
            jax.ShapeDtypeStruct((b, 1, 128), jnp.float32),
        ],
        compiler_params=pltpu.CompilerParams(
            dimension_semantics=("parallel",)),
    )(x, embedding)

    quantized_st = jnp.transpose(q.reshape(b, h, w, c), (0, 3, 1, 2))
    enc_idx = idx.reshape(b, h, w)
    loss_out = loss[:, 0, 0]
    return (quantized_st, enc_idx, loss_out)
